# baseline (speedup 1.0000x reference)
import functools

import jax
import jax.numpy as jnp
from jax import lax
from jax.experimental import pallas as pl
from jax.experimental.pallas import tpu as pltpu
from jax.experimental.pallas import tpu_sc as plsc

_NC = 2
_NS = 16
_NW = _NC * _NS

_BT = 128
_BI = 1024
_BH = 1024


def _routing_metadata(top_k_index, top_k_weights, T, K, E, Pmax, NB):
    A = T * K
    e_flat = top_k_index.reshape(A).astype(jnp.int32)
    w_flat = top_k_weights.reshape(A).astype(jnp.float32)
    onehot = (e_flat[:, None] == jnp.arange(E, dtype=jnp.int32)[None, :]).astype(jnp.int32)
    rank = jnp.take_along_axis(jnp.cumsum(onehot, axis=0) - onehot,
                               e_flat[:, None], axis=1)[:, 0]
    counts = jnp.sum(onehot, axis=0)
    blocks_per_e = (counts + _BT - 1) // _BT
    block_end = jnp.cumsum(blocks_per_e)
    group_start = (block_end - blocks_per_e) * _BT
    pos = group_start[e_flat] + rank
    tok = jnp.arange(A, dtype=jnp.int32) // K
    src_tok = jnp.zeros((Pmax,), jnp.int32).at[pos].set(tok)
    w_row = jnp.zeros((Pmax,), jnp.float32).at[pos].set(w_flat)
    gid = jnp.searchsorted(block_end, jnp.arange(NB), side="right")
    gid = jnp.minimum(gid, E - 1).astype(jnp.int32)
    return pos.astype(jnp.int32), src_tok, w_row, gid


def _sc_gather(table, src_tok, Pmax, W):
    rows_per_w = Pmax // _NW
    CH = 32
    n_chunks = rows_per_w // CH
    mesh = plsc.VectorSubcoreMesh(core_axis_name="c", subcore_axis_name="s")

    @functools.partial(
        pl.kernel, mesh=mesh,
        out_type=jax.ShapeDtypeStruct((Pmax, W), jnp.int32),
        scratch_types=[
            pltpu.VMEM((CH,), jnp.int32),
            pltpu.VMEM((CH, W), jnp.int32),
            pltpu.SemaphoreType.DMA,
        ],
    )
    def gather_k(hs_hbm, idx_hbm, out_hbm, idx_v, rows_v, sem):
        wid = lax.axis_index("s") * _NC + lax.axis_index("c")
        base = wid * rows_per_w

        def body(i, carry):
            off = base + i * CH
            pltpu.sync_copy(idx_hbm.at[pl.ds(off, CH)], idx_v)
            pltpu.async_copy(hs_hbm.at[idx_v], rows_v, sem).wait()
            pltpu.sync_copy(rows_v, out_hbm.at[pl.ds(off, CH)])
            return carry

        lax.fori_loop(0, n_chunks, body, 0)

    return gather_k(table, src_tok)


def _sc_combine(out_rows, pos, T, K, H):
    toks_per_w = T // _NW
    CT = 16
    n_chunks = toks_per_w // CT
    mesh = plsc.VectorSubcoreMesh(core_axis_name="c", subcore_axis_name="s")

    @functools.partial(
        pl.kernel, mesh=mesh,
        out_type=jax.ShapeDtypeStruct((T, H), jnp.float32),
        scratch_types=[
            pltpu.VMEM((K * CT,), jnp.int32),
            pltpu.VMEM((K * CT, H), jnp.float32),
            pltpu.VMEM((CT, H), jnp.float32),
            pltpu.SemaphoreType.DMA,
        ],
    )
    def combine_k(rows_hbm, pos_hbm, out_hbm, idx_v, rows_v, acc_v, sem):
        wid = lax.axis_index("s") * _NC + lax.axis_index("c")
        base = wid * toks_per_w

        def body(i, carry):
            toff = base + i * CT
            pltpu.sync_copy(pos_hbm.at[pl.ds(K * toff, K * CT)], idx_v)
            pltpu.async_copy(rows_hbm.at[idx_v], rows_v, sem).wait()

            def add_body(j, c):
                sl = pl.ds(j * 16, 16)
                for t in range(CT):
                    acc_v[t, sl] = rows_v[K * t, sl] + rows_v[K * t + 1, sl]
                return c

            lax.fori_loop(0, H // 16, add_body, 0)
            pltpu.sync_copy(acc_v, out_hbm.at[pl.ds(toff, CT)])
            return carry

        lax.fori_loop(0, n_chunks, body, 0)

    return combine_k(out_rows, pos)


def _mlp_gate_up(xs, Wg, Wu, gid, Pmax, H, I, NB):
    def body(gid_ref, xs_ref, wg_ref, wu_ref, hs_ref):
        x = xs_ref[...]
        dn = (((1,), (1,)), ((), ()))
        g = lax.dot_general(x, wg_ref[0], dn, preferred_element_type=jnp.float32)
        u = lax.dot_general(x, wu_ref[0], dn, preferred_element_type=jnp.float32)
        hs_ref[...] = ((g * jax.nn.sigmoid(g)) * u).astype(jnp.bfloat16)

    grid = (I // _BI, NB)
    spec = pltpu.PrefetchScalarGridSpec(
        num_scalar_prefetch=1,
        grid=grid,
        in_specs=[
            pl.BlockSpec((_BT, H), lambda ib, nb, gid_ref: (nb, 0)),
            pl.BlockSpec((1, _BI, H), lambda ib, nb, gid_ref: (gid_ref[nb], ib, 0)),
            pl.BlockSpec((1, _BI, H), lambda ib, nb, gid_ref: (gid_ref[nb], ib, 0)),
        ],
        out_specs=pl.BlockSpec((_BT, _BI), lambda ib, nb, gid_ref: (nb, ib)),
    )
    return pl.pallas_call(
        body,
        grid_spec=spec,
        out_shape=jax.ShapeDtypeStruct((Pmax, I), jnp.bfloat16),
        compiler_params=pltpu.CompilerParams(
            dimension_semantics=("arbitrary", "arbitrary")),
    )(gid, xs, Wg, Wu)


def _mlp_down(hs, Wd, w_row3, gid, Pmax, H, I, NB):
    def body(gid_ref, hs_ref, wd_ref, ws_ref, out_ref):
        dn = (((1,), (1,)), ((), ()))
        o = lax.dot_general(hs_ref[...], wd_ref[0], dn,
                            preferred_element_type=jnp.float32)
        out_ref[...] = o * ws_ref[0, 0][:, None]

    grid = (H // _BH, NB)
    spec = pltpu.PrefetchScalarGridSpec(
        num_scalar_prefetch=1,
        grid=grid,
        in_specs=[
            pl.BlockSpec((_BT, I), lambda hb, nb, gid_ref: (nb, 0)),
            pl.BlockSpec((1, _BH, I), lambda hb, nb, gid_ref: (gid_ref[nb], hb, 0)),
            pl.BlockSpec((1, 1, _BT), lambda hb, nb, gid_ref: (nb, 0, 0)),
        ],
        out_specs=pl.BlockSpec((_BT, _BH), lambda hb, nb, gid_ref: (nb, hb)),
    )
    return pl.pallas_call(
        body,
        grid_spec=spec,
        out_shape=jax.ShapeDtypeStruct((Pmax, H), jnp.float32),
        compiler_params=pltpu.CompilerParams(
            dimension_semantics=("arbitrary", "arbitrary")),
    )(gid, hs, Wd, w_row3)


def kernel(hidden_states, top_k_index, top_k_weights, Wg, Wu, Wd):
    T, H = hidden_states.shape
    K = top_k_index.shape[1]
    E, I, _ = Wg.shape
    Pmax = T * K + E * _BT
    NB = Pmax // _BT

    pos, src_tok, w_row, gid = _routing_metadata(
        top_k_index, top_k_weights, T, K, E, Pmax, NB)

    h32 = lax.bitcast_convert_type(
        hidden_states.astype(jnp.bfloat16).reshape(T, H // 2, 2), jnp.int32)
    xs32 = _sc_gather(h32, src_tok, Pmax, H // 2)
    xs = lax.bitcast_convert_type(xs32, jnp.bfloat16).reshape(Pmax, H)

    hs = _mlp_gate_up(xs, Wg, Wu, gid, Pmax, H, I, NB)
    out_rows = _mlp_down(hs, Wd, w_row.reshape(NB, 1, _BT), gid, Pmax, H, I, NB)
    return _sc_combine(out_rows, pos, T, K, H)


# speedup vs baseline: 1.2955x; 1.2955x over previous
import functools

import jax
import jax.numpy as jnp
from jax import lax
from jax.experimental import pallas as pl
from jax.experimental.pallas import tpu as pltpu
from jax.experimental.pallas import tpu_sc as plsc

_NC = 2
_NS = 16
_NW = _NC * _NS

_BT = 128
_BI = 1024
_BH = 1024


def _routing_metadata(top_k_index, top_k_weights, T, K, E, Pmax, NB):
    A = T * K
    e_flat = top_k_index.reshape(A).astype(jnp.int32)
    w_flat = top_k_weights.reshape(A).astype(jnp.float32)
    onehot = (e_flat[:, None] == jnp.arange(E, dtype=jnp.int32)[None, :]).astype(jnp.int32)
    rank = jnp.take_along_axis(jnp.cumsum(onehot, axis=0) - onehot,
                               e_flat[:, None], axis=1)[:, 0]
    counts = jnp.sum(onehot, axis=0)
    blocks_per_e = (counts + _BT - 1) // _BT
    block_end = jnp.cumsum(blocks_per_e)
    group_start = (block_end - blocks_per_e) * _BT
    pos = group_start[e_flat] + rank
    tok = jnp.arange(A, dtype=jnp.int32) // K
    src_tok = jnp.zeros((Pmax,), jnp.int32).at[pos].set(tok)
    w_row = jnp.zeros((Pmax,), jnp.float32).at[pos].set(w_flat)
    gid = jnp.searchsorted(block_end, jnp.arange(NB), side="right")
    gid = jnp.minimum(gid, E - 1).astype(jnp.int32)
    return pos.astype(jnp.int32), src_tok, w_row, gid


def _sc_gather(table, src_tok, Pmax, W):
    rows_per_w = Pmax // _NW
    CH = 32
    n_chunks = rows_per_w // CH
    mesh = plsc.VectorSubcoreMesh(core_axis_name="c", subcore_axis_name="s")

    @functools.partial(
        pl.kernel, mesh=mesh,
        out_type=jax.ShapeDtypeStruct((Pmax, W), jnp.float32),
        scratch_types=[
            pltpu.VMEM((CH,), jnp.int32),
            pltpu.VMEM((CH, W), jnp.float32),
            pltpu.SemaphoreType.DMA,
        ],
    )
    def gather_k(hs_hbm, idx_hbm, out_hbm, idx_v, rows_v, sem):
        wid = lax.axis_index("s") * _NC + lax.axis_index("c")
        base = wid * rows_per_w

        def body(i, carry):
            off = base + i * CH
            pltpu.sync_copy(idx_hbm.at[pl.ds(off, CH)], idx_v)
            pltpu.async_copy(hs_hbm.at[idx_v], rows_v, sem).wait()
            pltpu.sync_copy(rows_v, out_hbm.at[pl.ds(off, CH)])
            return carry

        lax.fori_loop(0, n_chunks, body, 0)

    return gather_k(table, src_tok)


def _sc_combine(out_rows, pos, T, K, H):
    toks_per_w = T // _NW
    CT = 16
    n_chunks = toks_per_w // CT
    mesh = plsc.VectorSubcoreMesh(core_axis_name="c", subcore_axis_name="s")

    @functools.partial(
        pl.kernel, mesh=mesh,
        out_type=jax.ShapeDtypeStruct((T, H), jnp.float32),
        scratch_types=[
            pltpu.VMEM((K * CT,), jnp.int32),
            pltpu.VMEM((K * CT, H), jnp.float32),
            pltpu.VMEM((CT, H), jnp.float32),
            pltpu.SemaphoreType.DMA,
        ],
    )
    def combine_k(rows_hbm, pos_hbm, out_hbm, idx_v, rows_v, acc_v, sem):
        wid = lax.axis_index("s") * _NC + lax.axis_index("c")
        base = wid * toks_per_w

        def body(i, carry):
            toff = base + i * CT
            pltpu.sync_copy(pos_hbm.at[pl.ds(K * toff, K * CT)], idx_v)
            pltpu.async_copy(rows_hbm.at[idx_v], rows_v, sem).wait()

            def add_body(j, c):
                sl = pl.ds(j * 16, 16)
                for t in range(CT):
                    acc_v[t, sl] = rows_v[K * t, sl] + rows_v[K * t + 1, sl]
                return c

            lax.fori_loop(0, H // 16, add_body, 0)
            pltpu.sync_copy(acc_v, out_hbm.at[pl.ds(toff, CT)])
            return carry

        lax.fori_loop(0, n_chunks, body, 0)

    return combine_k(out_rows, pos)


def _mlp_gate_up(xs, Wg, Wu, gid, Pmax, H, I, NB):
    def body(gid_ref, xs_ref, wg_ref, wu_ref, hs_ref):
        x = xs_ref[...]
        dn = (((1,), (1,)), ((), ()))
        g = lax.dot_general(x, wg_ref[0], dn, preferred_element_type=jnp.float32)
        u = lax.dot_general(x, wu_ref[0], dn, preferred_element_type=jnp.float32)
        hs_ref[...] = ((g * jax.nn.sigmoid(g)) * u).astype(jnp.bfloat16)

    grid = (I // _BI, NB)
    spec = pltpu.PrefetchScalarGridSpec(
        num_scalar_prefetch=1,
        grid=grid,
        in_specs=[
            pl.BlockSpec((_BT, H), lambda ib, nb, gid_ref: (nb, 0)),
            pl.BlockSpec((1, _BI, H), lambda ib, nb, gid_ref: (gid_ref[nb], ib, 0)),
            pl.BlockSpec((1, _BI, H), lambda ib, nb, gid_ref: (gid_ref[nb], ib, 0)),
        ],
        out_specs=pl.BlockSpec((_BT, _BI), lambda ib, nb, gid_ref: (nb, ib)),
    )
    return pl.pallas_call(
        body,
        grid_spec=spec,
        out_shape=jax.ShapeDtypeStruct((Pmax, I), jnp.bfloat16),
        compiler_params=pltpu.CompilerParams(
            dimension_semantics=("arbitrary", "arbitrary")),
    )(gid, xs, Wg, Wu)


def _mlp_down(hs, Wd, w_row3, gid, Pmax, H, I, NB):
    def body(gid_ref, hs_ref, wd_ref, ws_ref, out_ref):
        dn = (((1,), (1,)), ((), ()))
        o = lax.dot_general(hs_ref[...], wd_ref[0], dn,
                            preferred_element_type=jnp.float32)
        out_ref[...] = o * ws_ref[0, 0][:, None]

    grid = (H // _BH, NB)
    spec = pltpu.PrefetchScalarGridSpec(
        num_scalar_prefetch=1,
        grid=grid,
        in_specs=[
            pl.BlockSpec((_BT, I), lambda hb, nb, gid_ref: (nb, 0)),
            pl.BlockSpec((1, _BH, I), lambda hb, nb, gid_ref: (gid_ref[nb], hb, 0)),
            pl.BlockSpec((1, 1, _BT), lambda hb, nb, gid_ref: (nb, 0, 0)),
        ],
        out_specs=pl.BlockSpec((_BT, _BH), lambda hb, nb, gid_ref: (nb, hb)),
    )
    return pl.pallas_call(
        body,
        grid_spec=spec,
        out_shape=jax.ShapeDtypeStruct((Pmax, H), jnp.float32),
        compiler_params=pltpu.CompilerParams(
            dimension_semantics=("arbitrary", "arbitrary")),
    )(gid, hs, Wd, w_row3)


def kernel(hidden_states, top_k_index, top_k_weights, Wg, Wu, Wd):
    T, H = hidden_states.shape
    K = top_k_index.shape[1]
    E, I, _ = Wg.shape
    Pmax = T * K + E * _BT
    NB = Pmax // _BT

    pos, src_tok, w_row, gid = _routing_metadata(
        top_k_index, top_k_weights, T, K, E, Pmax, NB)

    xs = _sc_gather(hidden_states, src_tok, Pmax, H)
    hs = _mlp_gate_up(xs, Wg, Wu, gid, Pmax, H, I, NB)
    out_rows = _mlp_down(hs, Wd, w_row.reshape(NB, 1, _BT), gid, Pmax, H, I, NB)
    return _sc_combine(out_rows, pos, T, K, H)


# speedup vs baseline: 1.9568x; 1.5105x over previous
import functools

import jax
import jax.numpy as jnp
from jax import lax
from jax.experimental import pallas as pl
from jax.experimental.pallas import tpu as pltpu
from jax.experimental.pallas import tpu_sc as plsc

_NC = 2
_NS = 16
_NW = _NC * _NS

_BT = 256
_BI = 1024
_BH = 1024


def _routing_metadata(top_k_index, top_k_weights, T, K, E, Pmax, NB):
    A = T * K
    e_flat = top_k_index.reshape(A).astype(jnp.int32)
    w_flat = top_k_weights.reshape(A).astype(jnp.float32)
    onehot = (e_flat[:, None] == jnp.arange(E, dtype=jnp.int32)[None, :]).astype(jnp.int32)
    rank = jnp.take_along_axis(jnp.cumsum(onehot, axis=0) - onehot,
                               e_flat[:, None], axis=1)[:, 0]
    counts = jnp.sum(onehot, axis=0)
    blocks_per_e = (counts + _BT - 1) // _BT
    block_end = jnp.cumsum(blocks_per_e)
    group_start = (block_end - blocks_per_e) * _BT
    pos = group_start[e_flat] + rank
    tok = jnp.arange(A, dtype=jnp.int32) // K
    src_tok = jnp.zeros((Pmax,), jnp.int32).at[pos].set(tok)
    w_row = jnp.zeros((Pmax,), jnp.float32).at[pos].set(w_flat)
    gid = jnp.searchsorted(block_end, jnp.arange(NB), side="right")
    gid = jnp.minimum(gid, E - 1).astype(jnp.int32)
    return pos.astype(jnp.int32), src_tok, w_row, gid


def _sc_gather(table, src_tok, Pmax, W):
    rows_per_w = Pmax // _NW
    CH = 32
    n_chunks = rows_per_w // CH
    mesh = plsc.VectorSubcoreMesh(core_axis_name="c", subcore_axis_name="s")

    @functools.partial(
        pl.kernel, mesh=mesh,
        out_type=jax.ShapeDtypeStruct((Pmax, W), jnp.float32),
        scratch_types=[
            pltpu.VMEM((CH,), jnp.int32),
            pltpu.VMEM((CH, W), jnp.float32),
            pltpu.SemaphoreType.DMA,
        ],
    )
    def gather_k(hs_hbm, idx_hbm, out_hbm, idx_v, rows_v, sem):
        wid = lax.axis_index("s") * _NC + lax.axis_index("c")
        base = wid * rows_per_w

        def body(i, carry):
            off = base + i * CH
            pltpu.sync_copy(idx_hbm.at[pl.ds(off, CH)], idx_v)
            pltpu.async_copy(hs_hbm.at[idx_v], rows_v, sem).wait()
            pltpu.sync_copy(rows_v, out_hbm.at[pl.ds(off, CH)])
            return carry

        lax.fori_loop(0, n_chunks, body, 0)

    return gather_k(table, src_tok)


def _sc_combine(out_rows, pos, T, K, H):
    toks_per_w = T // _NW
    CT = 16
    n_chunks = toks_per_w // CT
    mesh = plsc.VectorSubcoreMesh(core_axis_name="c", subcore_axis_name="s")

    @functools.partial(
        pl.kernel, mesh=mesh,
        out_type=jax.ShapeDtypeStruct((T, H), jnp.float32),
        scratch_types=[
            pltpu.VMEM((K * CT,), jnp.int32),
            pltpu.VMEM((K * CT, H), jnp.float32),
            pltpu.VMEM((CT, H), jnp.float32),
            pltpu.SemaphoreType.DMA,
        ],
    )
    def combine_k(rows_hbm, pos_hbm, out_hbm, idx_v, rows_v, acc_v, sem):
        wid = lax.axis_index("s") * _NC + lax.axis_index("c")
        base = wid * toks_per_w

        def body(i, carry):
            toff = base + i * CT
            pltpu.sync_copy(pos_hbm.at[pl.ds(K * toff, K * CT)], idx_v)
            pltpu.async_copy(rows_hbm.at[idx_v], rows_v, sem).wait()

            def add_body(j, c):
                sl = pl.ds(j * 16, 16)
                for t in range(CT):
                    acc_v[t, sl] = rows_v[K * t, sl] + rows_v[K * t + 1, sl]
                return c

            lax.fori_loop(0, H // 16, add_body, 0)
            pltpu.sync_copy(acc_v, out_hbm.at[pl.ds(toff, CT)])
            return carry

        lax.fori_loop(0, n_chunks, body, 0)

    return combine_k(out_rows, pos)


def _mlp_gate_up(xs, Wg, Wu, gid, Pmax, H, I, NB):
    def body(gid_ref, xs_ref, wg_ref, wu_ref, hs_ref):
        x = xs_ref[...]
        dn = (((1,), (1,)), ((), ()))
        g = lax.dot_general(x, wg_ref[0], dn, preferred_element_type=jnp.float32)
        u = lax.dot_general(x, wu_ref[0], dn, preferred_element_type=jnp.float32)
        hs_ref[...] = ((g * jax.nn.sigmoid(g)) * u).astype(jnp.bfloat16)

    grid = (I // _BI, NB)
    spec = pltpu.PrefetchScalarGridSpec(
        num_scalar_prefetch=1,
        grid=grid,
        in_specs=[
            pl.BlockSpec((_BT, H), lambda ib, nb, gid_ref: (nb, 0)),
            pl.BlockSpec((1, _BI, H), lambda ib, nb, gid_ref: (gid_ref[nb], ib, 0)),
            pl.BlockSpec((1, _BI, H), lambda ib, nb, gid_ref: (gid_ref[nb], ib, 0)),
        ],
        out_specs=pl.BlockSpec((_BT, _BI), lambda ib, nb, gid_ref: (nb, ib)),
    )
    return pl.pallas_call(
        body,
        grid_spec=spec,
        out_shape=jax.ShapeDtypeStruct((Pmax, I), jnp.bfloat16),
        compiler_params=pltpu.CompilerParams(
            dimension_semantics=("arbitrary", "arbitrary")),
    )(gid, xs, Wg, Wu)


def _mlp_down(hs, Wd, w_row3, gid, Pmax, H, I, NB):
    def body(gid_ref, hs_ref, wd_ref, ws_ref, out_ref):
        dn = (((1,), (1,)), ((), ()))
        o = lax.dot_general(hs_ref[...], wd_ref[0], dn,
                            preferred_element_type=jnp.float32)
        out_ref[...] = o * ws_ref[0, 0][:, None]

    grid = (H // _BH, NB)
    spec = pltpu.PrefetchScalarGridSpec(
        num_scalar_prefetch=1,
        grid=grid,
        in_specs=[
            pl.BlockSpec((_BT, I), lambda hb, nb, gid_ref: (nb, 0)),
            pl.BlockSpec((1, _BH, I), lambda hb, nb, gid_ref: (gid_ref[nb], hb, 0)),
            pl.BlockSpec((1, 1, _BT), lambda hb, nb, gid_ref: (nb, 0, 0)),
        ],
        out_specs=pl.BlockSpec((_BT, _BH), lambda hb, nb, gid_ref: (nb, hb)),
    )
    return pl.pallas_call(
        body,
        grid_spec=spec,
        out_shape=jax.ShapeDtypeStruct((Pmax, H), jnp.float32),
        compiler_params=pltpu.CompilerParams(
            dimension_semantics=("arbitrary", "arbitrary")),
    )(gid, hs, Wd, w_row3)


def kernel(hidden_states, top_k_index, top_k_weights, Wg, Wu, Wd):
    T, H = hidden_states.shape
    K = top_k_index.shape[1]
    E, I, _ = Wg.shape
    Pmax = T * K + E * _BT
    NB = Pmax // _BT

    pos, src_tok, w_row, gid = _routing_metadata(
        top_k_index, top_k_weights, T, K, E, Pmax, NB)

    xs = _sc_gather(hidden_states, src_tok, Pmax, H)
    hs = _mlp_gate_up(xs, Wg, Wu, gid, Pmax, H, I, NB)
    out_rows = _mlp_down(hs, Wd, w_row.reshape(NB, 1, _BT), gid, Pmax, H, I, NB)
    return _sc_combine(out_rows, pos, T, K, H)
